# Initial kernel scaffold; baseline (speedup 1.0000x reference)
import jax
import jax.numpy as jnp
from jax.experimental import pallas as pl


def kernel(hidden_states, query, router_W, in_W, in_b, out_W, out_b, memory):
    raise NotImplementedError("write your pallas kernel here")


# speedup vs baseline: 2.2655x; 2.2655x over previous
import functools
import math

import jax
import jax.numpy as jnp
from jax.experimental import pallas as pl


def _moe_body(hs_ref, q_ref, rwt_ref, inwt_ref, inb_ref, outwt_ref,
              outb_ref, mem_ref, memt_ref,
              out_ref, disp_ref, prob_ref, loss_ref,
              *, n_tokens, n_experts, scale):
    i = pl.program_id(0)
    nsteps = pl.num_programs(0)
    T = hs_ref.shape[0]
    E = n_experts

    logits = jnp.dot(hs_ref[...].astype(jnp.bfloat16),
                     rwt_ref[...].astype(jnp.bfloat16),
                     preferred_element_type=jnp.float32)
    w = jax.nn.softmax(logits, axis=-1)
    eids = jax.lax.broadcasted_iota(jnp.int32, (T, E), 1)
    w1 = jnp.max(w, axis=-1, keepdims=True)
    i1 = jnp.argmax(w, axis=-1).reshape(T, 1)
    wm = jnp.where(eids == i1, -jnp.inf, w)
    w2 = jnp.max(wm, axis=-1, keepdims=True)
    i2 = jnp.argmax(wm, axis=-1).reshape(T, 1)
    denom = w1 + w2 + 1e-8
    tw1 = w1 / denom
    tw2 = w2 / denom

    disp_part = jnp.sum((eids == i1).astype(jnp.float32), axis=0,
                        keepdims=True)
    prob_part = jnp.sum(w, axis=0, keepdims=True)

    @pl.when(i == 0)
    def _init():
        disp_ref[...] = jnp.zeros_like(disp_ref)
        prob_ref[...] = jnp.zeros_like(prob_ref)

    disp_ref[...] += disp_part
    prob_ref[...] += prob_part

    @pl.when(i == nsteps - 1)
    def _finalize():
        df = disp_ref[...] / n_tokens
        pf = prob_ref[...] / n_tokens
        loss_ref[...] = (E * jnp.sum(df * pf)).reshape(1, 1)

    mq = jnp.dot(q_ref[...].astype(jnp.bfloat16),
                 inwt_ref[...].astype(jnp.bfloat16),
                 preferred_element_type=jnp.float32) + inb_ref[...]
    mqb = mq.astype(jnp.bfloat16)

    acc = jnp.zeros(mq.shape, jnp.float32)
    for e in range(E):
        attn = jnp.dot(mqb, memt_ref[e].astype(jnp.bfloat16),
                       preferred_element_type=jnp.float32) * scale
        aw = jax.nn.softmax(attn, axis=-1)
        eo = jnp.dot(aw.astype(jnp.bfloat16),
                     mem_ref[e].astype(jnp.bfloat16),
                     preferred_element_type=jnp.float32)
        we = tw1 * (i1 == e).astype(jnp.float32) \
            + tw2 * (i2 == e).astype(jnp.float32)
        acc = acc + we * eo

    out_ref[...] = jnp.dot(acc.astype(jnp.bfloat16),
                           outwt_ref[...].astype(jnp.bfloat16),
                           preferred_element_type=jnp.float32) + outb_ref[...]


def kernel(hidden_states, query, router_W, in_W, in_b, out_W, out_b, memory):
    B, S, D = hidden_states.shape
    E = router_W.shape[0]
    DM = in_W.shape[0]
    C = memory.shape[1]
    N = B * S
    T = 512 if N % 512 == 0 else N
    scale = 1.0 / math.sqrt(DM)

    hs2 = hidden_states.reshape(N, D)
    q2 = query.reshape(N, D)
    rwt = router_W.T
    inwt = in_W.T
    outwt = out_W.T
    memt = memory.transpose(0, 2, 1)
    inb2 = in_b.reshape(1, DM)
    outb2 = out_b.reshape(1, D)

    grid = (N // T,)
    body = functools.partial(_moe_body, n_tokens=float(N), n_experts=E,
                             scale=scale)
    out, _, _, loss = pl.pallas_call(
        body,
        grid=grid,
        in_specs=[
            pl.BlockSpec((T, D), lambda i: (i, 0)),
            pl.BlockSpec((T, D), lambda i: (i, 0)),
            pl.BlockSpec((D, E), lambda i: (0, 0)),
            pl.BlockSpec((D, DM), lambda i: (0, 0)),
            pl.BlockSpec((1, DM), lambda i: (0, 0)),
            pl.BlockSpec((DM, D), lambda i: (0, 0)),
            pl.BlockSpec((1, D), lambda i: (0, 0)),
            pl.BlockSpec((E, C, DM), lambda i: (0, 0, 0)),
            pl.BlockSpec((E, DM, C), lambda i: (0, 0, 0)),
        ],
        out_specs=[
            pl.BlockSpec((T, D), lambda i: (i, 0)),
            pl.BlockSpec((1, E), lambda i: (0, 0)),
            pl.BlockSpec((1, E), lambda i: (0, 0)),
            pl.BlockSpec((1, 1), lambda i: (0, 0)),
        ],
        out_shape=[
            jax.ShapeDtypeStruct((N, D), jnp.float32),
            jax.ShapeDtypeStruct((1, E), jnp.float32),
            jax.ShapeDtypeStruct((1, E), jnp.float32),
            jax.ShapeDtypeStruct((1, 1), jnp.float32),
        ],
    )(hs2, q2, rwt, inwt, inb2, outwt, outb2, memory, memt)

    return (out.reshape(B, S, D), loss.reshape(()))


# speedup vs baseline: 2.8974x; 1.2789x over previous
import functools
import math

import jax
import jax.numpy as jnp
from jax.experimental import pallas as pl


def _moe_body(hs_ref, q_ref, rwt_ref, inwt_ref, inb_ref, outwt_ref,
              outb_ref, mem_ref, memt_ref,
              out_ref, disp_ref, prob_ref, loss_ref,
              *, n_tokens, n_experts, scale):
    i = pl.program_id(0)
    nsteps = pl.num_programs(0)
    T = hs_ref.shape[0]
    E = n_experts

    logits = jnp.dot(hs_ref[...].astype(jnp.bfloat16),
                     rwt_ref[...].astype(jnp.bfloat16),
                     preferred_element_type=jnp.float32)
    w = jax.nn.softmax(logits, axis=-1)
    eids = jax.lax.broadcasted_iota(jnp.int32, (T, E), 1)
    w1 = jnp.max(w, axis=-1, keepdims=True)
    i1 = jnp.argmax(w, axis=-1).reshape(T, 1)
    wm = jnp.where(eids == i1, -jnp.inf, w)
    w2 = jnp.max(wm, axis=-1, keepdims=True)
    i2 = jnp.argmax(wm, axis=-1).reshape(T, 1)
    denom = w1 + w2 + 1e-8
    tw1 = w1 / denom
    tw2 = w2 / denom

    disp_part = jnp.sum((eids == i1).astype(jnp.float32), axis=0,
                        keepdims=True)
    prob_part = jnp.sum(w, axis=0, keepdims=True)

    @pl.when(i == 0)
    def _init():
        disp_ref[...] = jnp.zeros_like(disp_ref)
        prob_ref[...] = jnp.zeros_like(prob_ref)

    disp_ref[...] += disp_part
    prob_ref[...] += prob_part

    @pl.when(i == nsteps - 1)
    def _finalize():
        df = disp_ref[...] / n_tokens
        pf = prob_ref[...] / n_tokens
        loss_ref[...] = (E * jnp.sum(df * pf)).reshape(1, 1)

    mq = jnp.dot(q_ref[...].astype(jnp.bfloat16),
                 inwt_ref[...].astype(jnp.bfloat16),
                 preferred_element_type=jnp.float32) + inb_ref[...]
    log2e = 1.4426950408889634
    mqs = (mq * (scale * log2e)).astype(jnp.bfloat16)

    acc = jnp.zeros(mq.shape, jnp.float32)
    for e in range(E):
        attn = jnp.dot(mqs, memt_ref[e].astype(jnp.bfloat16),
                       preferred_element_type=jnp.float32)
        p = jnp.exp2(attn)
        s = jnp.sum(p, axis=-1, keepdims=True)
        eo = jnp.dot(p.astype(jnp.bfloat16),
                     mem_ref[e].astype(jnp.bfloat16),
                     preferred_element_type=jnp.float32)
        we = (tw1 * (i1 == e).astype(jnp.float32)
              + tw2 * (i2 == e).astype(jnp.float32)) / s
        acc = acc + we * eo

    out_ref[...] = jnp.dot(acc.astype(jnp.bfloat16),
                           outwt_ref[...].astype(jnp.bfloat16),
                           preferred_element_type=jnp.float32) + outb_ref[...]


def kernel(hidden_states, query, router_W, in_W, in_b, out_W, out_b, memory):
    B, S, D = hidden_states.shape
    E = router_W.shape[0]
    DM = in_W.shape[0]
    C = memory.shape[1]
    N = B * S
    T = 512 if N % 512 == 0 else N
    scale = 1.0 / math.sqrt(DM)

    hs2 = hidden_states.reshape(N, D)
    q2 = query.reshape(N, D)
    rwt = router_W.T
    inwt = in_W.T
    outwt = out_W.T
    memt = memory.transpose(0, 2, 1)
    inb2 = in_b.reshape(1, DM)
    outb2 = out_b.reshape(1, D)

    grid = (N // T,)
    body = functools.partial(_moe_body, n_tokens=float(N), n_experts=E,
                             scale=scale)
    out, _, _, loss = pl.pallas_call(
        body,
        grid=grid,
        in_specs=[
            pl.BlockSpec((T, D), lambda i: (i, 0)),
            pl.BlockSpec((T, D), lambda i: (i, 0)),
            pl.BlockSpec((D, E), lambda i: (0, 0)),
            pl.BlockSpec((D, DM), lambda i: (0, 0)),
            pl.BlockSpec((1, DM), lambda i: (0, 0)),
            pl.BlockSpec((DM, D), lambda i: (0, 0)),
            pl.BlockSpec((1, D), lambda i: (0, 0)),
            pl.BlockSpec((E, C, DM), lambda i: (0, 0, 0)),
            pl.BlockSpec((E, DM, C), lambda i: (0, 0, 0)),
        ],
        out_specs=[
            pl.BlockSpec((T, D), lambda i: (i, 0)),
            pl.BlockSpec((1, E), lambda i: (0, 0)),
            pl.BlockSpec((1, E), lambda i: (0, 0)),
            pl.BlockSpec((1, 1), lambda i: (0, 0)),
        ],
        out_shape=[
            jax.ShapeDtypeStruct((N, D), jnp.float32),
            jax.ShapeDtypeStruct((1, E), jnp.float32),
            jax.ShapeDtypeStruct((1, E), jnp.float32),
            jax.ShapeDtypeStruct((1, 1), jnp.float32),
        ],
    )(hs2, q2, rwt, inwt, inb2, outwt, outb2, memory, memt)

    return (out.reshape(B, S, D), loss.reshape(()))


# speedup vs baseline: 2.9445x; 1.0163x over previous
import functools
import math

import jax
import jax.numpy as jnp
from jax.experimental import pallas as pl


def _moe_body(hs_ref, q_ref, rwt_ref, inwt_ref, inb_ref, outwt_ref,
              outb_ref, mem_ref, memt_ref,
              out_ref, disp_ref, prob_ref, loss_ref,
              *, n_tokens, n_experts, scale):
    i = pl.program_id(0)
    nsteps = pl.num_programs(0)
    T = hs_ref.shape[0]
    E = n_experts

    logits = jnp.dot(hs_ref[...].astype(jnp.bfloat16),
                     rwt_ref[...].astype(jnp.bfloat16),
                     preferred_element_type=jnp.float32)
    w = jax.nn.softmax(logits, axis=-1)
    eids = jax.lax.broadcasted_iota(jnp.int32, (T, E), 1)
    w1 = jnp.max(w, axis=-1, keepdims=True)
    i1 = jnp.argmax(w, axis=-1).reshape(T, 1)
    wm = jnp.where(eids == i1, -jnp.inf, w)
    w2 = jnp.max(wm, axis=-1, keepdims=True)
    i2 = jnp.argmax(wm, axis=-1).reshape(T, 1)
    denom = w1 + w2 + 1e-8
    tw1 = w1 / denom
    tw2 = w2 / denom

    disp_part = jnp.sum((eids == i1).astype(jnp.float32), axis=0,
                        keepdims=True)
    prob_part = jnp.sum(w, axis=0, keepdims=True)

    @pl.when(i == 0)
    def _init():
        disp_ref[...] = jnp.zeros_like(disp_ref)
        prob_ref[...] = jnp.zeros_like(prob_ref)

    disp_ref[...] += disp_part
    prob_ref[...] += prob_part

    @pl.when(i == nsteps - 1)
    def _finalize():
        df = disp_ref[...] / n_tokens
        pf = prob_ref[...] / n_tokens
        loss_ref[...] = (E * jnp.sum(df * pf)).reshape(1, 1)

    mq = jnp.dot(q_ref[...].astype(jnp.bfloat16),
                 inwt_ref[...].astype(jnp.bfloat16),
                 preferred_element_type=jnp.float32) + inb_ref[...]
    log2e = 1.4426950408889634
    mqs = (mq * (scale * log2e)).astype(jnp.bfloat16)

    acc = jnp.zeros(mq.shape, jnp.float32)
    for e in range(E):
        attn = jnp.dot(mqs, memt_ref[e].astype(jnp.bfloat16),
                       preferred_element_type=jnp.float32)
        p = jnp.exp2(attn)
        s = jnp.sum(p, axis=-1, keepdims=True)
        eo = jnp.dot(p.astype(jnp.bfloat16),
                     mem_ref[e].astype(jnp.bfloat16),
                     preferred_element_type=jnp.float32)
        we = (tw1 * (i1 == e).astype(jnp.float32)
              + tw2 * (i2 == e).astype(jnp.float32)) / s
        acc = acc + we * eo

    out_ref[...] = jnp.dot(acc.astype(jnp.bfloat16),
                           outwt_ref[...].astype(jnp.bfloat16),
                           preferred_element_type=jnp.float32) + outb_ref[...]


def kernel(hidden_states, query, router_W, in_W, in_b, out_W, out_b, memory):
    B, S, D = hidden_states.shape
    E = router_W.shape[0]
    DM = in_W.shape[0]
    C = memory.shape[1]
    N = B * S
    T = 1024 if N % 1024 == 0 else N
    scale = 1.0 / math.sqrt(DM)

    hs2 = hidden_states.reshape(N, D)
    q2 = query.reshape(N, D)
    rwt = router_W.T
    inwt = in_W.T
    outwt = out_W.T
    memt = memory.transpose(0, 2, 1)
    inb2 = in_b.reshape(1, DM)
    outb2 = out_b.reshape(1, D)

    grid = (N // T,)
    body = functools.partial(_moe_body, n_tokens=float(N), n_experts=E,
                             scale=scale)
    out, _, _, loss = pl.pallas_call(
        body,
        grid=grid,
        in_specs=[
            pl.BlockSpec((T, D), lambda i: (i, 0)),
            pl.BlockSpec((T, D), lambda i: (i, 0)),
            pl.BlockSpec((D, E), lambda i: (0, 0)),
            pl.BlockSpec((D, DM), lambda i: (0, 0)),
            pl.BlockSpec((1, DM), lambda i: (0, 0)),
            pl.BlockSpec((DM, D), lambda i: (0, 0)),
            pl.BlockSpec((1, D), lambda i: (0, 0)),
            pl.BlockSpec((E, C, DM), lambda i: (0, 0, 0)),
            pl.BlockSpec((E, DM, C), lambda i: (0, 0, 0)),
        ],
        out_specs=[
            pl.BlockSpec((T, D), lambda i: (i, 0)),
            pl.BlockSpec((1, E), lambda i: (0, 0)),
            pl.BlockSpec((1, E), lambda i: (0, 0)),
            pl.BlockSpec((1, 1), lambda i: (0, 0)),
        ],
        out_shape=[
            jax.ShapeDtypeStruct((N, D), jnp.float32),
            jax.ShapeDtypeStruct((1, E), jnp.float32),
            jax.ShapeDtypeStruct((1, E), jnp.float32),
            jax.ShapeDtypeStruct((1, 1), jnp.float32),
        ],
    )(hs2, q2, rwt, inwt, inb2, outwt, outb2, memory, memt)

    return (out.reshape(B, S, D), loss.reshape(()))
